# baseline (speedup 1.0000x reference)
import jax
import jax.numpy as jnp
from jax.experimental import pallas as pl
from jax.experimental.pallas import tpu as pltpu

_EPS = 1e-5
_N = 32768
_B = 16
_G = 4
_TQ = 2048
_TOK = _G * _TQ
_NB = _N // _TOK
_NPASS = 4
_NF = float(_N)


def _body(x_ref, len_ref, w1_ref, w2_ref, w3_ref, w4_ref, w5_ref,
          g1_ref, be1_ref, g2_ref, be2_ref, g3_ref, be3_ref,
          g4_ref, be4_ref, g5_ref, be5_ref,
          out_ref,
          s1, q1, s2, q2, s3, q3, s4, q4, s5, q5,
          sc1, sh1, sc2, sh2, sc3, sh3, sc4, sh4, sc5, sh5,
          mm, mr, dn, nm):
    p = pl.program_id(0)
    b = pl.program_id(1)
    xb = x_ref[...]

    def dot(a, w):
        return jnp.dot(a, w, preferred_element_type=jnp.float32)

    ones_col = jnp.ones((_TQ, 1), jnp.float32)

    def rowsum(h):
        return jax.lax.dot_general(ones_col, h, (((0,), (0,)), ((), ())),
                                   preferred_element_type=jnp.float32,
                                   precision=jax.lax.Precision.HIGHEST)

    def moments(h, s_acc, q_acc):
        s_acc[...] += rowsum(h)
        q_acc[...] += rowsum(h * h)

    def gsum(v, c):
        return (v[:, 0 * c:1 * c] + v[:, 1 * c:2 * c]
                + v[:, 2 * c:3 * c] + v[:, 3 * c:4 * c])

    def fold(s_acc, q_acc, g_ref, be_ref, sc_acc, sh_acc, c=None, tile=True):
        if c is None:
            s, q = s_acc[...], q_acc[...]
        else:
            s, q = gsum(s_acc[...], c), gsum(q_acc[...], c)
        m = s / _NF
        v = q / _NF - m * m
        sc = g_ref[...] * jax.lax.rsqrt(v + _EPS)
        sh = be_ref[...] - m * sc
        if c is not None and tile:
            sc = jnp.concatenate([sc] * _G, axis=1)
            sh = jnp.concatenate([sh] * _G, axis=1)
        sc_acc[...] = sc
        sh_acc[...] = sh

    @pl.when((p == 0) & (b == 0))
    def _init():
        for r in (s1, q1, s2, q2, s3, q3, s4, q4, s5, q5, dn, nm):
            r[...] = jnp.zeros_like(r)
        mm[...] = jnp.full_like(mm, -jnp.inf)

    def xg(g):
        return xb[:, 32 * g:32 * (g + 1)]

    @pl.when(p == 0)
    def _p0():
        moments(dot(xb, w1_ref[...]), s1, q1)
        for g in range(_G):
            moments(dot(xg(g), w4_ref[...]), s4, q4)

    @pl.when((p == 1) & (b == 0))
    def _fold14():
        fold(s1, q1, g1_ref, be1_ref, sc1, sh1, c=16)
        fold(s4, q4, g4_ref, be4_ref, sc4, sh4)

    def h1of(xb):
        return jnp.maximum(dot(xb, w1_ref[...]) * sc1[...] + sh1[...], 0.0)

    def h4of(g):
        return jnp.maximum(dot(xg(g), w4_ref[...]) * sc4[...] + sh4[...], 0.0)

    @pl.when(p == 1)
    def _p1():
        moments(dot(h1of(xb), w2_ref[...]), s2, q2)
        for g in range(_G):
            moments(dot(h4of(g), w5_ref[...]), s5, q5)

    @pl.when((p == 2) & (b == 0))
    def _fold25():
        fold(s2, q2, g2_ref, be2_ref, sc2, sh2, c=8)
        fold(s5, q5, g5_ref, be5_ref, sc5, sh5)

    def h2of(xb):
        return jnp.maximum(dot(h1of(xb), w2_ref[...]) * sc2[...] + sh2[...],
                           0.0)

    def segbounds():
        lens = len_ref[...]
        si = jax.lax.broadcasted_iota(jnp.int32, (_B, _B), 0)
        sj = jax.lax.broadcasted_iota(jnp.int32, (_B, _B), 1)
        lens_col = jnp.sum(jnp.where(sj == si, lens, 0), axis=1, keepdims=True)
        starts = jnp.sum(jnp.where(si < sj, lens_col, 0), axis=0, keepdims=True)
        return lens, starts

    def maskof(b, g, lens, starts):
        pos = (_G * jax.lax.broadcasted_iota(jnp.int32, (_TQ, _B), 0)
               + (_G * b * _TQ + g))
        return (pos >= starts) & (pos < starts + lens)

    @pl.when(p == 2)
    def _p2():
        pre3 = dot(h2of(xb), w3_ref[...])
        moments(pre3, s3, q3)
        lens, starts = segbounds()
        acc = mm[...]
        for g in range(_G):
            mask = maskof(b, g, lens, starts)
            blk = jnp.max(jnp.where(mask, pre3[:, g:g + 1], -jnp.inf),
                          axis=0, keepdims=True)
            acc = jnp.maximum(acc, blk)
        mm[...] = acc

    @pl.when((p == 3) & (b == 0))
    def _fold3():
        fold(s3, q3, g3_ref, be3_ref, sc3, sh3, c=1, tile=False)
        mr[...] = jnp.maximum(mm[...] * sc3[0, 0] + sh3[0, 0], 0.0)

    @pl.when(p == 3)
    def _p3():
        o1 = jnp.maximum(dot(h2of(xb), w3_ref[...]) * sc3[0, 0] + sh3[0, 0],
                         0.0)
        lens, starts = segbounds()
        dacc = dn[...]
        nacc = nm[...]
        for g in range(_G):
            l5raw = dot(h4of(g), w5_ref[...])
            mask = maskof(b, g, lens, starts)
            mcol = jnp.sum(jnp.where(mask, mr[...], 0.0), axis=1,
                           keepdims=True)
            e = jnp.exp(o1[:, g:g + 1] - mcol)
            me = jnp.where(mask, e, 0.0)
            dacc += rowsum(me)
            nacc += jax.lax.dot_general(me, l5raw, (((0,), (0,)), ((), ())),
                                        preferred_element_type=jnp.float32,
                                        precision=jax.lax.Precision.HIGHEST)
        dn[...] = dacc
        nm[...] = nacc

    @pl.when((p == 3) & (b == _NB - 1))
    def _fin():
        lens = len_ref[...].astype(jnp.float32)
        crow = 1.0 / (dn[...] * lens)
        si = jax.lax.broadcasted_iota(jnp.int32, (_B, _B), 0)
        sj = jax.lax.broadcasted_iota(jnp.int32, (_B, _B), 1)
        ccol = jnp.sum(jnp.where(sj == si, crow, 0.0), axis=1, keepdims=True)
        dncol = jnp.sum(jnp.where(sj == si, dn[...], 0.0), axis=1,
                        keepdims=True)
        res = (nm[...] * sc5[...] + dncol * sh5[...]) * ccol
        norm = jnp.sqrt(jnp.sum(res * res, axis=1, keepdims=True))
        out_ref[...] = res / jnp.maximum(norm, 1e-12)


def kernel(x, length, W1, b1, g1, be1, W2, b2, g2, be2, W3, b3, g3, be3,
           W4, b4, g4, be4, W5, b5, g5, be5):
    row = lambda v: v.reshape(1, -1).astype(jnp.float32)
    len2 = length.astype(jnp.int32).reshape(1, _B)
    x2 = x.reshape(_N // _G, _G * 32)
    eye = jnp.eye(_G, dtype=jnp.float32)
    bd = lambda W: jnp.kron(eye, W.T)
    f32 = jnp.float32
    full = lambda shape: pl.BlockSpec(shape, lambda p, b: (0, 0))
    in_specs = [pl.BlockSpec((_TQ, _G * 32), lambda p, b: (b, 0)),
                full((1, _B))]
    args = [x2, len2]
    for W, packed in ((W1, True), (W2, True), (W3, True),
                      (W4, False), (W5, False)):
        wt = bd(W) if packed else W.T
        args.append(wt)
        in_specs.append(full(wt.shape))
    for g, be in ((g1, be1), (g2, be2), (g3, be3), (g4, be4), (g5, be5)):
        args += [row(g), row(be)]
        in_specs += [full((1, g.shape[0]))] * 2
    ch = lambda c: pltpu.VMEM((1, c), f32)
    return pl.pallas_call(
        _body,
        grid=(_NPASS, _NB),
        in_specs=in_specs,
        out_specs=full((_B, 128)),
        out_shape=jax.ShapeDtypeStruct((_B, 128), f32),
        scratch_shapes=[
            ch(64), ch(64), ch(32), ch(32), ch(_G), ch(_G),
            ch(64), ch(64), ch(128), ch(128),
            ch(64), ch(64), ch(32), ch(32), ch(1), ch(1),
            ch(64), ch(64), ch(128), ch(128),
            ch(_B), ch(_B), ch(_B), pltpu.VMEM((_B, 128), f32),
        ],
    )(*args)


# speedup vs baseline: 1.5269x; 1.5269x over previous
import jax
import jax.numpy as jnp
from jax.experimental import pallas as pl
from jax.experimental.pallas import tpu as pltpu

_EPS = 1e-5
_N = 32768
_B = 16
_G = 4
_TQ = 2048
_TOK = _G * _TQ
_NB = _N // _TOK
_NPASS = 4
_NF = float(_N)


def _body(x_ref, len_ref, w1_ref, w2_ref, w3_ref, w4_ref, w5_ref,
          g1_ref, be1_ref, g2_ref, be2_ref, g3_ref, be3_ref,
          g4_ref, be4_ref, g5_ref, be5_ref,
          out_ref,
          s1, q1, s2, q2, s3, q3, s4, q4, s5, q5,
          sc1, sh1, sc2, sh2, sc3, sh3, sc4, sh4, sc5, sh5,
          mm, mr, dn, nm):
    p = pl.program_id(0)
    b = pl.program_id(1)
    xb = x_ref[...]

    def dot(a, w):
        return jnp.dot(a, w, preferred_element_type=jnp.float32)

    def moments(h, s_acc, q_acc):
        s_acc[...] += jnp.sum(h, axis=0, keepdims=True)
        q_acc[...] += jnp.sum(h * h, axis=0, keepdims=True)

    def gsum(v, c):
        return (v[:, 0 * c:1 * c] + v[:, 1 * c:2 * c]
                + v[:, 2 * c:3 * c] + v[:, 3 * c:4 * c])

    def fold(s_acc, q_acc, g_ref, be_ref, sc_acc, sh_acc, c=None, tile=True):
        if c is None:
            s, q = s_acc[...], q_acc[...]
        else:
            s, q = gsum(s_acc[...], c), gsum(q_acc[...], c)
        m = s / _NF
        v = q / _NF - m * m
        sc = g_ref[...] * jax.lax.rsqrt(v + _EPS)
        sh = be_ref[...] - m * sc
        if c is not None and tile:
            sc = jnp.concatenate([sc] * _G, axis=1)
            sh = jnp.concatenate([sh] * _G, axis=1)
        sc_acc[...] = sc
        sh_acc[...] = sh

    @pl.when((p == 0) & (b == 0))
    def _init():
        for r in (s1, q1, s2, q2, s3, q3, s4, q4, s5, q5, dn, nm):
            r[...] = jnp.zeros_like(r)
        mm[...] = jnp.full_like(mm, -jnp.inf)

    def xg(g):
        return xb[:, 32 * g:32 * (g + 1)]

    @pl.when(p == 0)
    def _p0():
        moments(dot(xb, w1_ref[...]), s1, q1)
        for g in range(_G):
            moments(dot(xg(g), w4_ref[...]), s4, q4)

    @pl.when((p == 1) & (b == 0))
    def _fold14():
        fold(s1, q1, g1_ref, be1_ref, sc1, sh1, c=16)
        fold(s4, q4, g4_ref, be4_ref, sc4, sh4)

    def h1of(xb):
        return jnp.maximum(dot(xb, w1_ref[...]) * sc1[...] + sh1[...], 0.0)

    def h4of(g):
        return jnp.maximum(dot(xg(g), w4_ref[...]) * sc4[...] + sh4[...], 0.0)

    @pl.when(p == 1)
    def _p1():
        moments(dot(h1of(xb), w2_ref[...]), s2, q2)
        for g in range(_G):
            moments(dot(h4of(g), w5_ref[...]), s5, q5)

    @pl.when((p == 2) & (b == 0))
    def _fold25():
        fold(s2, q2, g2_ref, be2_ref, sc2, sh2, c=8)
        fold(s5, q5, g5_ref, be5_ref, sc5, sh5)

    def h2of(xb):
        return jnp.maximum(dot(h1of(xb), w2_ref[...]) * sc2[...] + sh2[...],
                           0.0)

    def segbounds():
        lens = len_ref[...]
        si = jax.lax.broadcasted_iota(jnp.int32, (_B, _B), 0)
        sj = jax.lax.broadcasted_iota(jnp.int32, (_B, _B), 1)
        lens_col = jnp.sum(jnp.where(sj == si, lens, 0), axis=1, keepdims=True)
        starts = jnp.sum(jnp.where(si < sj, lens_col, 0), axis=0, keepdims=True)
        return lens, starts

    def maskof(b, g, lens, starts):
        pos = (_G * jax.lax.broadcasted_iota(jnp.int32, (_TQ, _B), 0)
               + (_G * b * _TQ + g))
        return (pos >= starts) & (pos < starts + lens)

    @pl.when(p == 2)
    def _p2():
        pre3 = dot(h2of(xb), w3_ref[...])
        moments(pre3, s3, q3)
        lens, starts = segbounds()
        acc = mm[...]
        for g in range(_G):
            mask = maskof(b, g, lens, starts)
            blk = jnp.max(jnp.where(mask, pre3[:, g:g + 1], -jnp.inf),
                          axis=0, keepdims=True)
            acc = jnp.maximum(acc, blk)
        mm[...] = acc

    @pl.when((p == 3) & (b == 0))
    def _fold3():
        fold(s3, q3, g3_ref, be3_ref, sc3, sh3, c=1, tile=False)
        mr[...] = jnp.maximum(mm[...] * sc3[0, 0] + sh3[0, 0], 0.0)

    @pl.when(p == 3)
    def _p3():
        o1 = jnp.maximum(dot(h2of(xb), w3_ref[...]) * sc3[0, 0] + sh3[0, 0],
                         0.0)
        lens, starts = segbounds()
        dacc = dn[...]
        nacc = nm[...]
        for g in range(_G):
            l5raw = dot(h4of(g), w5_ref[...])
            mask = maskof(b, g, lens, starts)
            mcol = jnp.sum(jnp.where(mask, mr[...], 0.0), axis=1,
                           keepdims=True)
            e = jnp.exp(o1[:, g:g + 1] - mcol)
            me = jnp.where(mask, e, 0.0)
            dacc += jnp.sum(me, axis=0, keepdims=True)
            nacc += jax.lax.dot_general(me, l5raw, (((0,), (0,)), ((), ())),
                                        preferred_element_type=jnp.float32,
                                        precision=jax.lax.Precision.HIGHEST)
        dn[...] = dacc
        nm[...] = nacc

    @pl.when((p == 3) & (b == _NB - 1))
    def _fin():
        lens = len_ref[...].astype(jnp.float32)
        crow = 1.0 / (dn[...] * lens)
        si = jax.lax.broadcasted_iota(jnp.int32, (_B, _B), 0)
        sj = jax.lax.broadcasted_iota(jnp.int32, (_B, _B), 1)
        ccol = jnp.sum(jnp.where(sj == si, crow, 0.0), axis=1, keepdims=True)
        dncol = jnp.sum(jnp.where(sj == si, dn[...], 0.0), axis=1,
                        keepdims=True)
        res = (nm[...] * sc5[...] + dncol * sh5[...]) * ccol
        norm = jnp.sqrt(jnp.sum(res * res, axis=1, keepdims=True))
        out_ref[...] = res / jnp.maximum(norm, 1e-12)


def kernel(x, length, W1, b1, g1, be1, W2, b2, g2, be2, W3, b3, g3, be3,
           W4, b4, g4, be4, W5, b5, g5, be5):
    row = lambda v: v.reshape(1, -1).astype(jnp.float32)
    len2 = length.astype(jnp.int32).reshape(1, _B)
    x2 = x.reshape(_N // _G, _G * 32)
    eye = jnp.eye(_G, dtype=jnp.float32)
    bd = lambda W: jnp.kron(eye, W.T)
    f32 = jnp.float32
    full = lambda shape: pl.BlockSpec(shape, lambda p, b: (0, 0))
    in_specs = [pl.BlockSpec((_TQ, _G * 32), lambda p, b: (b, 0)),
                full((1, _B))]
    args = [x2, len2]
    for W, packed in ((W1, True), (W2, True), (W3, True),
                      (W4, False), (W5, False)):
        wt = bd(W) if packed else W.T
        args.append(wt)
        in_specs.append(full(wt.shape))
    for g, be in ((g1, be1), (g2, be2), (g3, be3), (g4, be4), (g5, be5)):
        args += [row(g), row(be)]
        in_specs += [full((1, g.shape[0]))] * 2
    ch = lambda c: pltpu.VMEM((1, c), f32)
    return pl.pallas_call(
        _body,
        grid=(_NPASS, _NB),
        in_specs=in_specs,
        out_specs=full((_B, 128)),
        out_shape=jax.ShapeDtypeStruct((_B, 128), f32),
        scratch_shapes=[
            ch(64), ch(64), ch(32), ch(32), ch(_G), ch(_G),
            ch(64), ch(64), ch(128), ch(128),
            ch(64), ch(64), ch(32), ch(32), ch(1), ch(1),
            ch(64), ch(64), ch(128), ch(128),
            ch(_B), ch(_B), ch(_B), pltpu.VMEM((_B, 128), f32),
        ],
    )(*args)


# speedup vs baseline: 1.6294x; 1.0671x over previous
import jax
import jax.numpy as jnp
from jax.experimental import pallas as pl
from jax.experimental.pallas import tpu as pltpu

_EPS = 1e-5
_N = 32768
_B = 16
_G = 4
_TQ = 2048
_TOK = _G * _TQ
_NB = _N // _TOK
_NPASS = 4
_NF = float(_N)


def _body(x_ref, len_ref, w1_ref, w2_ref, w3_ref, w4_ref, w5_ref,
          g1_ref, be1_ref, g2_ref, be2_ref, g3_ref, be3_ref,
          g4_ref, be4_ref, g5_ref, be5_ref,
          out_ref,
          s1, q1, s2, q2, s3, q3, s4, q4, s5, q5,
          sc1, sh1, sc2, sh2, sc3, sh3, sc4, sh4, sc5, sh5,
          mm, mr, dn, nm):
    p = pl.program_id(0)
    b = pl.program_id(1)
    xb = x_ref[...]

    def dot(a, w):
        return jnp.dot(a, w, preferred_element_type=jnp.float32)

    def moments(h, s_acc, q_acc):
        s_acc[...] += jnp.sum(h, axis=0, keepdims=True)
        q_acc[...] += jnp.sum(h * h, axis=0, keepdims=True)

    def gsum(v, c):
        return (v[:, 0 * c:1 * c] + v[:, 1 * c:2 * c]
                + v[:, 2 * c:3 * c] + v[:, 3 * c:4 * c])

    def fold(s_acc, q_acc, g_ref, be_ref, sc_acc, sh_acc, c=None, tile=True):
        if c is None:
            s, q = s_acc[...], q_acc[...]
        else:
            s, q = gsum(s_acc[...], c), gsum(q_acc[...], c)
        m = s / _NF
        v = q / _NF - m * m
        sc = g_ref[...] * jax.lax.rsqrt(v + _EPS)
        sh = be_ref[...] - m * sc
        if c is not None and tile:
            sc = jnp.concatenate([sc] * _G, axis=1)
            sh = jnp.concatenate([sh] * _G, axis=1)
        sc_acc[...] = sc
        sh_acc[...] = sh

    @pl.when((p == 0) & (b == 0))
    def _init():
        for r in (s1, q1, s2, q2, s3, q3, s4, q4, s5, q5, dn, nm):
            r[...] = jnp.zeros_like(r)
        mm[...] = jnp.full_like(mm, -jnp.inf)

    def xg(g):
        return xb[:, 32 * g:32 * (g + 1)]

    @pl.when(p == 0)
    def _p0():
        moments(dot(xb, w1_ref[...]), s1, q1)
        for g in range(_G):
            moments(dot(xg(g), w4_ref[...]), s4, q4)

    @pl.when((p == 1) & (b == 0))
    def _fold14():
        fold(s1, q1, g1_ref, be1_ref, sc1, sh1, c=16)
        fold(s4, q4, g4_ref, be4_ref, sc4, sh4)

    def h1of(xb):
        return jnp.maximum(dot(xb, w1_ref[...]) * sc1[...] + sh1[...], 0.0)

    def h4of(g):
        return jnp.maximum(dot(xg(g), w4_ref[...]) * sc4[...] + sh4[...], 0.0)

    @pl.when(p == 1)
    def _p1():
        moments(dot(h1of(xb), w2_ref[...]), s2, q2)
        for g in range(_G):
            moments(dot(h4of(g), w5_ref[...]), s5, q5)

    @pl.when((p == 2) & (b == 0))
    def _fold25():
        fold(s2, q2, g2_ref, be2_ref, sc2, sh2, c=8)
        fold(s5, q5, g5_ref, be5_ref, sc5, sh5)

    def h2of(xb):
        return jnp.maximum(dot(h1of(xb), w2_ref[...]) * sc2[...] + sh2[...],
                           0.0)

    def segbounds():
        lens = len_ref[...]
        si = jax.lax.broadcasted_iota(jnp.int32, (_B, _B), 0)
        sj = jax.lax.broadcasted_iota(jnp.int32, (_B, _B), 1)
        lens_col = jnp.sum(jnp.where(sj == si, lens, 0), axis=1, keepdims=True)
        starts = jnp.sum(jnp.where(si < sj, lens_col, 0), axis=0, keepdims=True)
        return lens, starts

    def maskof(b, g, lens, starts):
        pos = (_G * jax.lax.broadcasted_iota(jnp.int32, (_TQ, _B), 0)
               + (_G * b * _TQ + g))
        return (pos >= starts) & (pos < starts + lens)

    @pl.when(p == 2)
    def _p2():
        pre3 = dot(h2of(xb), w3_ref[...])
        moments(pre3, s3, q3)
        lens, starts = segbounds()
        acc = mm[...]
        for g in range(_G):
            mask = maskof(b, g, lens, starts)
            blk = jnp.max(jnp.where(mask, pre3[:, g:g + 1], -jnp.inf),
                          axis=0, keepdims=True)
            acc = jnp.maximum(acc, blk)
        mm[...] = acc

    @pl.when((p == 3) & (b == 0))
    def _fold3():
        fold(s3, q3, g3_ref, be3_ref, sc3, sh3, c=1, tile=False)
        mr[...] = jnp.maximum(mm[...] * sc3[0, 0] + sh3[0, 0], 0.0)

    @pl.when(p == 3)
    def _p3():
        o1 = jnp.maximum(dot(h2of(xb), w3_ref[...]) * sc3[0, 0] + sh3[0, 0],
                         0.0)
        lens, starts = segbounds()
        dacc = dn[...]
        nacc = nm[...]
        for g in range(_G):
            l5raw = dot(h4of(g), w5_ref[...])
            mask = maskof(b, g, lens, starts)
            mcol = jnp.sum(jnp.where(mask, mr[...], 0.0), axis=1,
                           keepdims=True)
            e = jnp.exp(o1[:, g:g + 1] - mcol)
            me = jnp.where(mask, e, 0.0)
            dacc += jnp.sum(me, axis=0, keepdims=True)
            nacc += jax.lax.dot_general(me, l5raw, (((0,), (0,)), ((), ())),
                                        preferred_element_type=jnp.float32)
        dn[...] = dacc
        nm[...] = nacc

    @pl.when((p == 3) & (b == _NB - 1))
    def _fin():
        lens = len_ref[...].astype(jnp.float32)
        crow = 1.0 / (dn[...] * lens)
        si = jax.lax.broadcasted_iota(jnp.int32, (_B, _B), 0)
        sj = jax.lax.broadcasted_iota(jnp.int32, (_B, _B), 1)
        ccol = jnp.sum(jnp.where(sj == si, crow, 0.0), axis=1, keepdims=True)
        dncol = jnp.sum(jnp.where(sj == si, dn[...], 0.0), axis=1,
                        keepdims=True)
        res = (nm[...] * sc5[...] + dncol * sh5[...]) * ccol
        norm = jnp.sqrt(jnp.sum(res * res, axis=1, keepdims=True))
        out_ref[...] = res / jnp.maximum(norm, 1e-12)


def kernel(x, length, W1, b1, g1, be1, W2, b2, g2, be2, W3, b3, g3, be3,
           W4, b4, g4, be4, W5, b5, g5, be5):
    row = lambda v: v.reshape(1, -1).astype(jnp.float32)
    len2 = length.astype(jnp.int32).reshape(1, _B)
    x2 = x.reshape(_N // _G, _G * 32)
    eye = jnp.eye(_G, dtype=jnp.float32)
    bd = lambda W: jnp.kron(eye, W.T)
    f32 = jnp.float32
    full = lambda shape: pl.BlockSpec(shape, lambda p, b: (0, 0))
    in_specs = [pl.BlockSpec((_TQ, _G * 32), lambda p, b: (b, 0)),
                full((1, _B))]
    args = [x2, len2]
    for W, packed in ((W1, True), (W2, True), (W3, True),
                      (W4, False), (W5, False)):
        wt = bd(W) if packed else W.T
        args.append(wt)
        in_specs.append(full(wt.shape))
    for g, be in ((g1, be1), (g2, be2), (g3, be3), (g4, be4), (g5, be5)):
        args += [row(g), row(be)]
        in_specs += [full((1, g.shape[0]))] * 2
    ch = lambda c: pltpu.VMEM((1, c), f32)
    return pl.pallas_call(
        _body,
        grid=(_NPASS, _NB),
        in_specs=in_specs,
        out_specs=full((_B, 128)),
        out_shape=jax.ShapeDtypeStruct((_B, 128), f32),
        scratch_shapes=[
            ch(64), ch(64), ch(32), ch(32), ch(_G), ch(_G),
            ch(64), ch(64), ch(128), ch(128),
            ch(64), ch(64), ch(32), ch(32), ch(1), ch(1),
            ch(64), ch(64), ch(128), ch(128),
            ch(_B), ch(_B), ch(_B), pltpu.VMEM((_B, 128), f32),
        ],
    )(*args)
